# baseline (speedup 1.0000x reference)
import functools

import jax
import jax.numpy as jnp
from jax import lax
from jax.experimental import pallas as pl
from jax.experimental.pallas import tpu as pltpu
from jax.experimental.pallas import tpu_sc as plsc

_NC = 2
_NS = 16
_NW = _NC * _NS
_CH = 96
_D = 128

_f32 = jnp.float32


def _cdiv(a, b):
    return -(-a // b)


def _sc_attn_body(nchunks, n_pad,
                  qdot_hbm, hdot_hbm, eedot_hbm, jidx_hbm, iidx_hbm, h_hbm,
                  shift_hbm, zrow_hbm, zden_hbm, msg_out, den_out,
                  jidx_v, iidx_v, ee_v, shift_v,
                  qA, hdA, exA, rowsA, siA,
                  qB, hdB, exB, rowsB, siB,
                  qC, hdC, exC, rowsC, siC,
                  msg_sh, den_sh,
                  semA, semB, semC, semSA, semSB, semSC, semZ):
    c = lax.axis_index("c")
    s = lax.axis_index("s")
    wid = s * _NC + c
    stripe = n_pad // _NS
    soff = s * stripe
    wpe = nchunks * _CH

    z1 = pltpu.async_copy(zrow_hbm.at[pl.ds(soff, stripe)],
                          msg_sh.at[pl.ds(soff, stripe)], semZ)
    z2 = pltpu.async_copy(zden_hbm.at[pl.ds(soff, stripe)],
                          den_sh.at[pl.ds(soff, stripe)], semZ)

    pltpu.sync_copy(jidx_hbm.at[pl.ds(wid * wpe, wpe)], jidx_v)
    pltpu.sync_copy(iidx_hbm.at[pl.ds(wid * wpe, wpe)], iidx_v)
    pltpu.sync_copy(eedot_hbm.at[pl.ds(wid * wpe, wpe)], ee_v)
    pltpu.sync_copy(shift_hbm, shift_v)
    sv = shift_v[pl.ds(0, 16)][0]

    nbuf = len(bufs := ((qA, hdA, exA, rowsA, siA, semA, semSA),
                        (qB, hdB, exB, rowsB, siB, semB, semSB),
                        (qC, hdC, exC, rowsC, siC, semC, semSC)))

    def issue_gathers(ci, buf):
        q_v, hd_v, _, rows_v, si_v, sem, _ = buf
        base = ci * _CH
        g0 = pltpu.async_copy(iidx_hbm.at[pl.ds(wid * wpe + base, _CH)],
                              si_v, sem)
        g1 = pltpu.async_copy(qdot_hbm.at[iidx_v.at[pl.ds(base, _CH)]],
                              q_v, sem)
        g2 = pltpu.async_copy(hdot_hbm.at[jidx_v.at[pl.ds(base, _CH)]],
                              hd_v, sem)
        g3 = pltpu.async_copy(h_hbm.at[jidx_v.at[pl.ds(base, _CH)]],
                              rows_v, sem)
        return (g0, g1, g2, g3)

    gathers = {}
    scatters = {}
    for ci in range(min(nbuf - 1, nchunks)):
        gathers[ci] = issue_gathers(ci, bufs[ci % nbuf])
    z1.wait()
    z2.wait()
    plsc.subcore_barrier()

    for ci in range(nchunks):
        p = ci % nbuf
        q_v, hd_v, ex_v, rows_v, si_v, _, semS = bufs[p]
        nx = ci + nbuf - 1
        if nx < nchunks:
            if ci >= 1:
                for sd in scatters.pop(ci - 1):
                    sd.wait()
            gathers[nx] = issue_gathers(nx, bufs[nx % nbuf])
        for g in gathers.pop(ci):
            g.wait()
        for k in range(_CH // 16):
            sl = pl.ds(k * 16, 16)
            ex_v[sl] = jnp.exp(q_v[sl] + hd_v[sl]
                               + ee_v[pl.ds(ci * _CH + k * 16, 16)] + sv)

        def rbody(g, _):
            ex16 = ex_v[pl.ds(g * 16, 16)]
            for j in range(16):
                r = g * 16 + j
                a = ex16[j]
                for k in range(_D // 16):
                    sl = pl.ds(k * 16, 16)
                    rows_v[r, sl] = rows_v[r, sl] * a
            return 0
        lax.fori_loop(0, _CH // 16, rbody, 0)
        s1 = pltpu.async_copy(ex_v, den_sh.at[si_v], semS, add=True)
        s2 = pltpu.async_copy(rows_v, msg_sh.at[si_v], semS, add=True)
        scatters[ci] = (s1, s2)

    for ci in sorted(scatters):
        for sd in scatters[ci]:
            sd.wait()
    plsc.subcore_barrier()

    pltpu.sync_copy(msg_sh.at[pl.ds(soff, stripe)],
                    msg_out.at[c, pl.ds(soff, stripe)])
    pltpu.sync_copy(den_sh.at[pl.ds(soff, stripe)],
                    den_out.at[c, pl.ds(soff, stripe)])


def _make_sc_attn(n_pad, e_pad):
    nchunks = e_pad // (_NW * _CH)
    mesh = plsc.VectorSubcoreMesh(core_axis_name="c", subcore_axis_name="s")
    buf = [
        pltpu.VMEM((_CH,), _f32),
        pltpu.VMEM((_CH,), _f32),
        pltpu.VMEM((_CH,), _f32),
        pltpu.VMEM((_CH, _D), _f32),
        pltpu.VMEM((_CH,), jnp.int32),
    ]
    return pl.kernel(
        functools.partial(_sc_attn_body, nchunks, n_pad),
        out_type=(
            jax.ShapeDtypeStruct((_NC, n_pad, _D), _f32),
            jax.ShapeDtypeStruct((_NC, n_pad), _f32),
        ),
        mesh=mesh,
        scratch_types=[
            pltpu.VMEM((nchunks * _CH,), jnp.int32),
            pltpu.VMEM((nchunks * _CH,), jnp.int32),
            pltpu.VMEM((nchunks * _CH,), _f32),
            pltpu.VMEM((16,), _f32),
        ] + buf * 3 + [
            pltpu.VMEM_SHARED((10240, _D), _f32),
            pltpu.VMEM_SHARED((10240,), _f32),
        ] + [pltpu.SemaphoreType.DMA] * 7,
    )


_RCH = 128


def _sc_roots_body(nchunks, n_pad,
                   didx_hbm, sidx_hbm, cntd_out, cnts_out,
                   jidx_v, iidx_v, ones_v, cntd_sh, cnts_sh, sem):
    c = lax.axis_index("c")
    s = lax.axis_index("s")
    wid = s * _NC + c
    stripe = n_pad // _NS
    nb = stripe // _RCH

    if True:
        for k in range(_RCH // 16):
            ones_v[pl.ds(k * 16, 16)] = jnp.zeros((16,), _f32)
        for b in range(nb):
            off = s * stripe + b * _RCH
            pltpu.sync_copy(ones_v, cntd_sh.at[pl.ds(off, _RCH)])
            pltpu.sync_copy(ones_v, cnts_sh.at[pl.ds(off, _RCH)])
        for k in range(_RCH // 16):
            ones_v[pl.ds(k * 16, 16)] = jnp.ones((16,), _f32)
        plsc.subcore_barrier()

        def chunk(ci, _):
            base = wid * (nchunks * _RCH) + ci * _RCH
            pltpu.sync_copy(didx_hbm.at[pl.ds(base, _RCH)], jidx_v)
            pltpu.sync_copy(sidx_hbm.at[pl.ds(base, _RCH)], iidx_v)
            pltpu.sync_copy(ones_v, cntd_sh.at[jidx_v], add=True)
            pltpu.sync_copy(ones_v, cnts_sh.at[iidx_v], add=True)
            return 0

        lax.fori_loop(0, nchunks, chunk, 0)
        plsc.subcore_barrier()
        for b in range(nb):
            off = s * stripe + b * _RCH
            pltpu.sync_copy(cntd_sh.at[pl.ds(off, _RCH)],
                            cntd_out.at[c, pl.ds(off, _RCH)])
            pltpu.sync_copy(cnts_sh.at[pl.ds(off, _RCH)],
                            cnts_out.at[c, pl.ds(off, _RCH)])


def _make_sc_roots(n_pad, e_pad):
    nchunks = e_pad // (_NW * _RCH)
    mesh = plsc.VectorSubcoreMesh(core_axis_name="c", subcore_axis_name="s")
    return pl.kernel(
        functools.partial(_sc_roots_body, nchunks, n_pad),
        out_type=(
            jax.ShapeDtypeStruct((_NC, n_pad), _f32),
            jax.ShapeDtypeStruct((_NC, n_pad), _f32),
        ),
        mesh=mesh,
        scratch_types=[
            pltpu.VMEM((_RCH,), jnp.int32),
            pltpu.VMEM((_RCH,), jnp.int32),
            pltpu.VMEM((_RCH,), _f32),
            pltpu.VMEM_SHARED((10240,), _f32),
            pltpu.VMEM_SHARED((10240,), _f32),
            pltpu.SemaphoreType.DMA,
        ],
    )


def _tc_edge_body(ea_ref, welT_ref, wak_ref, bel_ref, eedot_ref, eemax_ref):
    i = pl.program_id(0)
    wak = wak_ref[...]
    wtil = jnp.dot(welT_ref[...], wak, preferred_element_type=_f32)
    bv = jnp.dot(bel_ref[...], wak, preferred_element_type=_f32)
    ed = jnp.dot(ea_ref[...], wtil, preferred_element_type=_f32) + bv
    eedot_ref[...] = ed

    @pl.when(i == 0)
    def _():
        eemax_ref[0, 0] = -1e30
    eemax_ref[0, 0] = jnp.maximum(eemax_ref[0, 0], jnp.max(ed))


def _tc_edge(ea, welT, wak, bel, blk):
    e = ea.shape[0]
    grid = e // blk
    return pl.pallas_call(
        _tc_edge_body,
        grid=(grid,),
        in_specs=[
            pl.BlockSpec((blk, _D), lambda i: (i, 0)),
            pl.BlockSpec((_D, _D), lambda i: (0, 0)),
            pl.BlockSpec((_D, 1), lambda i: (0, 0)),
            pl.BlockSpec((1, _D), lambda i: (0, 0)),
        ],
        out_specs=[
            pl.BlockSpec((blk, 1), lambda i: (i, 0)),
            pl.BlockSpec((1, 1), lambda i: (0, 0), memory_space=pltpu.SMEM),
        ],
        out_shape=[
            jax.ShapeDtypeStruct((e, 1), _f32),
            jax.ShapeDtypeStruct((1, 1), _f32),
        ],
    )(ea, welT, wak, bel)


def _tc_sweep_body(hp_ref, wih_ref, bih_ref, bhh_ref, waq_ref, wak_ref,
                   cnt_ref, gi_ref, qdot_ref, qmax_ref, h0_ref, hdot_ref,
                   hmax_ref):
    i = pl.program_id(0)
    hp = hp_ref[...]
    gi = jnp.dot(hp, wih_ref[...], preferred_element_type=_f32) + bih_ref[...]
    gi_ref[...] = gi
    qd = jnp.dot(hp, waq_ref[...], preferred_element_type=_f32)
    qdot_ref[...] = qd
    bhh = bhh_ref[...]
    r = jax.nn.sigmoid(gi[:, :_D] + bhh[:, :_D])
    z = jax.nn.sigmoid(gi[:, _D:2 * _D] + bhh[:, _D:2 * _D])
    n = jnp.tanh(gi[:, 2 * _D:] + r * bhh[:, 2 * _D:])
    init = (1.0 - z) * n
    root = cnt_ref[...] == 0.0
    h0 = jnp.where(root, init, 0.0)
    h0_ref[...] = h0
    hd = jnp.dot(h0, wak_ref[...], preferred_element_type=_f32)
    hdot_ref[...] = hd

    @pl.when(i == 0)
    def _():
        qmax_ref[0, 0] = -1e30
        hmax_ref[0, 0] = -1e30
    qmax_ref[0, 0] = jnp.maximum(qmax_ref[0, 0], jnp.max(qd))
    hmax_ref[0, 0] = jnp.maximum(hmax_ref[0, 0], jnp.max(hd))


def _tc_sweep(hp, wihT, bih, bhh, waq, wak, cnt, blk):
    n = hp.shape[0]
    grid = n // blk
    return pl.pallas_call(
        _tc_sweep_body,
        grid=(grid,),
        in_specs=[
            pl.BlockSpec((blk, _D), lambda i: (i, 0)),
            pl.BlockSpec((_D, 3 * _D), lambda i: (0, 0)),
            pl.BlockSpec((1, 3 * _D), lambda i: (0, 0)),
            pl.BlockSpec((1, 3 * _D), lambda i: (0, 0)),
            pl.BlockSpec((_D, 1), lambda i: (0, 0)),
            pl.BlockSpec((_D, 1), lambda i: (0, 0)),
            pl.BlockSpec((blk, 1), lambda i: (i, 0)),
        ],
        out_specs=[
            pl.BlockSpec((blk, 3 * _D), lambda i: (i, 0)),
            pl.BlockSpec((blk, 1), lambda i: (i, 0)),
            pl.BlockSpec((1, 1), lambda i: (0, 0), memory_space=pltpu.SMEM),
            pl.BlockSpec((blk, _D), lambda i: (i, 0)),
            pl.BlockSpec((blk, 1), lambda i: (i, 0)),
            pl.BlockSpec((1, 1), lambda i: (0, 0), memory_space=pltpu.SMEM),
        ],
        out_shape=[
            jax.ShapeDtypeStruct((n, 3 * _D), _f32),
            jax.ShapeDtypeStruct((n, 1), _f32),
            jax.ShapeDtypeStruct((1, 1), _f32),
            jax.ShapeDtypeStruct((n, _D), _f32),
            jax.ShapeDtypeStruct((n, 1), _f32),
            jax.ShapeDtypeStruct((1, 1), _f32),
        ],
    )(hp, wihT, bih, bhh, waq, wak, cnt)


def _tc_level_body(gi_ref, m0_ref, m1_ref, d0_ref, d1_ref, hc_ref, whh_ref,
                   bhh_ref, wak_ref, h_ref, hdot_ref, hmax_ref):
    i = pl.program_id(0)
    d = d0_ref[...] + d1_ref[...]
    m = (m0_ref[...] + m1_ref[...]) / (d + 1e-16)
    gi = gi_ref[...]
    gh = jnp.dot(m, whh_ref[...], preferred_element_type=_f32) + bhh_ref[...]
    r = jax.nn.sigmoid(gi[:, :_D] + gh[:, :_D])
    z = jax.nn.sigmoid(gi[:, _D:2 * _D] + gh[:, _D:2 * _D])
    n = jnp.tanh(gi[:, 2 * _D:] + r * gh[:, 2 * _D:])
    upd = (1.0 - z) * n + z * m
    nm = d > 0.0
    h = jnp.where(nm, upd, hc_ref[...])
    h_ref[...] = h
    hd = jnp.dot(h, wak_ref[...], preferred_element_type=_f32)
    hdot_ref[...] = hd

    @pl.when(i == 0)
    def _():
        hmax_ref[0, 0] = -1e30
    hmax_ref[0, 0] = jnp.maximum(hmax_ref[0, 0], jnp.max(hd))


def _tc_level(gi, m0, m1, d0, d1, hc, whhT, bhh, wak, blk):
    n = hc.shape[0]
    grid = n // blk
    return pl.pallas_call(
        _tc_level_body,
        grid=(grid,),
        in_specs=[
            pl.BlockSpec((blk, 3 * _D), lambda i: (i, 0)),
            pl.BlockSpec((blk, _D), lambda i: (i, 0)),
            pl.BlockSpec((blk, _D), lambda i: (i, 0)),
            pl.BlockSpec((blk, 1), lambda i: (i, 0)),
            pl.BlockSpec((blk, 1), lambda i: (i, 0)),
            pl.BlockSpec((blk, _D), lambda i: (i, 0)),
            pl.BlockSpec((_D, 3 * _D), lambda i: (0, 0)),
            pl.BlockSpec((1, 3 * _D), lambda i: (0, 0)),
            pl.BlockSpec((_D, 1), lambda i: (0, 0)),
        ],
        out_specs=[
            pl.BlockSpec((blk, _D), lambda i: (i, 0)),
            pl.BlockSpec((blk, 1), lambda i: (i, 0)),
            pl.BlockSpec((1, 1), lambda i: (0, 0), memory_space=pltpu.SMEM),
        ],
        out_shape=[
            jax.ShapeDtypeStruct((n, _D), _f32),
            jax.ShapeDtypeStruct((n, 1), _f32),
            jax.ShapeDtypeStruct((1, 1), _f32),
        ],
    )(gi, m0, m1, d0, d1, hc, whhT, bhh, wak)


def _tc_relu_body(x_ref, o_ref):
    o_ref[...] = jnp.maximum(x_ref[...], 0.0)


def _tc_relu(x, blk):
    n = x.shape[0]
    return pl.pallas_call(
        _tc_relu_body,
        grid=(n // blk,),
        in_specs=[pl.BlockSpec((blk, _D), lambda i: (i, 0))],
        out_specs=pl.BlockSpec((blk, _D), lambda i: (i, 0)),
        out_shape=jax.ShapeDtypeStruct(x.shape, x.dtype),
    )(x)


def kernel(x, edge_index, edge_attr, edge_masks,
           Wel_f, bel_f, Wa_f, ba_f, Wih_f, Whh_f, bih_f, bhh_f,
           Wel_b, bel_b, Wa_b, ba_b, Wih_b, Whh_b, bih_b, bhh_b):
    n, d = x.shape
    e = edge_index.shape[1]
    nl = edge_masks.shape[0]
    el = e // nl

    n_pad = _cdiv(n, _NS * 128) * (_NS * 128)
    e_lvl = _cdiv(el, _NW * _CH) * (_NW * _CH)
    e_all = _cdiv(e, _NW * _RCH) * (_NW * _RCH)
    blk = n_pad // 10

    src = edge_index[0]
    dst = edge_index[1]

    pad_l = e_lvl - el
    src_l = jnp.pad(src.reshape(nl, el), ((0, 0), (0, pad_l)))
    dst_l = jnp.pad(dst.reshape(nl, el), ((0, 0), (0, pad_l)))
    zrow = jnp.zeros((n_pad, d), _f32)
    zden = jnp.zeros((n_pad,), _f32)

    pad_a = e_all - e
    src_a = jnp.pad(src, (0, pad_a), constant_values=n_pad - 1)
    dst_a = jnp.pad(dst, (0, pad_a), constant_values=n_pad - 1)

    x_pad = jnp.pad(x, ((0, n_pad - n), (0, 0)))
    e_blk = _cdiv(e, 1024) * 1024
    ea_pad = jnp.pad(edge_attr, ((0, e_blk - e), (0, 0)))

    sc_attn = _make_sc_attn(n_pad, e_lvl)
    sc_roots = _make_sc_roots(n_pad, e_all)

    cntd2, cnts2 = sc_roots(dst_a, src_a)
    cntd = (cntd2[0] + cntd2[1]).reshape(n_pad, 1)
    cnts = (cnts2[0] + cnts2[1]).reshape(n_pad, 1)

    def sweep(h_prev, cnt, j_l, i_l, order,
              Wel, bel, Wa, ba, Wih, Whh, bih, bhh):
        waq = Wa[0, :d].reshape(d, 1)
        wak = Wa[0, d:].reshape(d, 1)
        eedot, eemax = _tc_edge(ea_pad, Wel.T, wak, bel.reshape(1, d),
                                blk=1024)
        ee_l = jnp.pad(eedot[:e, 0].reshape(nl, el), ((0, 0), (0, pad_l)),
                       constant_values=-1e30)
        gi, qdot, qmax, h, hdot, hmax = _tc_sweep(
            h_prev, Wih.T, bih.reshape(1, 3 * d), bhh.reshape(1, 3 * d),
            waq, wak, cnt, blk)
        whhT = Whh.T
        bhh2 = bhh.reshape(1, 3 * d)
        for l in order:
            shift = ba[0] - qmax[0, 0] - hmax[0, 0] - eemax[0, 0]
            shift_v = jnp.full((16,), shift, _f32)
            msg2, den2 = sc_attn(qdot.reshape(n_pad), hdot.reshape(n_pad),
                                 ee_l[l], j_l[l], i_l[l], h, shift_v,
                                 zrow, zden)
            h, hdot, hmax = _tc_level(
                gi, msg2[0], msg2[1], den2[0].reshape(n_pad, 1),
                den2[1].reshape(n_pad, 1), h, whhT, bhh2, wak, blk)
        return h

    h_fwd = sweep(x_pad, cntd, src_l, dst_l, range(nl),
                  Wel_f, bel_f, Wa_f, ba_f, Wih_f, Whh_f, bih_f, bhh_f)
    h_bwd = sweep(h_fwd, cnts, dst_l, src_l, range(nl - 1, -1, -1),
                  Wel_b, bel_b, Wa_b, ba_b, Wih_b, Whh_b, bih_b, bhh_b)
    return _tc_relu(h_bwd, blk)[:n]


# speedup vs baseline: 1.3696x; 1.3696x over previous
import functools

import jax
import jax.numpy as jnp
from jax import lax
from jax.experimental import pallas as pl
from jax.experimental.pallas import tpu as pltpu
from jax.experimental.pallas import tpu_sc as plsc

_NC = 2
_NS = 16
_NW = _NC * _NS
_CH = 128
_D = 128

_f32 = jnp.float32


def _cdiv(a, b):
    return -(-a // b)


def _sc_attn_body(nchunks, n_pad,
                  qdot_hbm, hdot_hbm, eedot_hbm, jidx_hbm, iidx_hbm, h_hbm,
                  shift_hbm, zrow_hbm, zden_hbm, msg_out, den_out,
                  jidx_v, iidx_v, ee_v, shift_v,
                  qA, hdA, exA, rowsA, qB, hdB, exB, rowsB,
                  msg_sh, den_sh,
                  semA, semB, semSA, semSB, semZ):
    c = lax.axis_index("c")
    s = lax.axis_index("s")
    wid = s * _NC + c
    stripe = n_pad // _NS
    soff = s * stripe

    z1 = pltpu.async_copy(zrow_hbm.at[pl.ds(soff, stripe)],
                          msg_sh.at[pl.ds(soff, stripe)], semZ)
    z2 = pltpu.async_copy(zden_hbm.at[pl.ds(soff, stripe)],
                          den_sh.at[pl.ds(soff, stripe)], semZ)

    pltpu.sync_copy(jidx_hbm.at[wid], jidx_v)
    pltpu.sync_copy(iidx_hbm.at[wid], iidx_v)
    pltpu.sync_copy(eedot_hbm.at[pl.ds(wid * nchunks * _CH, nchunks * _CH)],
                    ee_v)
    pltpu.sync_copy(shift_hbm, shift_v)
    sv = shift_v[pl.ds(0, 16)][0]

    bufs = ((qA, hdA, exA, rowsA, semA, semSA),
            (qB, hdB, exB, rowsB, semB, semSB))

    def issue_gathers(ci, buf):
        q_v, hd_v, _, rows_v, sem, _ = buf
        g1 = pltpu.async_copy(qdot_hbm.at[iidx_v.at[ci]], q_v, sem)
        g2 = pltpu.async_copy(hdot_hbm.at[jidx_v.at[ci]], hd_v, sem)
        g3 = pltpu.async_copy(h_hbm.at[jidx_v.at[ci]], rows_v, sem)
        return (g1, g2, g3)

    gathers = [None, None]
    scatters = [None, None]
    gathers[0] = issue_gathers(0, bufs[0])
    z1.wait()
    z2.wait()
    plsc.subcore_barrier()

    for ci in range(nchunks):
        p = ci % 2
        q_v, hd_v, ex_v, rows_v, _, semS = bufs[p]
        if ci + 1 < nchunks:
            np_ = (ci + 1) % 2
            if scatters[np_] is not None:
                scatters[np_][0].wait()
                scatters[np_][1].wait()
                scatters[np_] = None
            gathers[np_] = issue_gathers(ci + 1, bufs[np_])
        for g in gathers[p]:
            g.wait()
        for k in range(_CH // 16):
            sl = pl.ds(k * 16, 16)
            ex_v[sl] = jnp.exp(q_v[sl] + hd_v[sl] + ee_v[pl.ds(ci * _CH + k * 16, 16)] + sv)

        def rbody(g, _):
            ex16 = ex_v[pl.ds(g * 16, 16)]
            for j in range(16):
                r = g * 16 + j
                a = ex16[j]
                for k in range(_D // 16):
                    sl = pl.ds(k * 16, 16)
                    rows_v[r, sl] = rows_v[r, sl] * a
            return 0
        lax.fori_loop(0, _CH // 16, rbody, 0)
        s1 = pltpu.async_copy(ex_v, den_sh.at[iidx_v.at[ci]], semS, add=True)
        s2 = pltpu.async_copy(rows_v, msg_sh.at[iidx_v.at[ci]], semS, add=True)
        scatters[p] = (s1, s2)

    for p in range(2):
        if scatters[p] is not None:
            scatters[p][0].wait()
            scatters[p][1].wait()
    plsc.subcore_barrier()

    pltpu.sync_copy(msg_sh.at[pl.ds(soff, stripe)],
                    msg_out.at[c, pl.ds(soff, stripe)])
    pltpu.sync_copy(den_sh.at[pl.ds(soff, stripe)],
                    den_out.at[c, pl.ds(soff, stripe)])


def _make_sc_attn(n_pad, e_pad):
    nchunks = e_pad // (_NW * _CH)
    mesh = plsc.VectorSubcoreMesh(core_axis_name="c", subcore_axis_name="s")
    return pl.kernel(
        functools.partial(_sc_attn_body, nchunks, n_pad),
        out_type=(
            jax.ShapeDtypeStruct((_NC, n_pad, _D), _f32),
            jax.ShapeDtypeStruct((_NC, n_pad), _f32),
        ),
        mesh=mesh,
        scratch_types=[
            pltpu.VMEM((nchunks, _CH), jnp.int32),
            pltpu.VMEM((nchunks, _CH), jnp.int32),
            pltpu.VMEM((nchunks * _CH,), _f32),
            pltpu.VMEM((16,), _f32),
            pltpu.VMEM((_CH,), _f32),
            pltpu.VMEM((_CH,), _f32),
            pltpu.VMEM((_CH,), _f32),
            pltpu.VMEM((_CH, _D), _f32),
            pltpu.VMEM((_CH,), _f32),
            pltpu.VMEM((_CH,), _f32),
            pltpu.VMEM((_CH,), _f32),
            pltpu.VMEM((_CH, _D), _f32),
            pltpu.VMEM_SHARED((10240, _D), _f32),
            pltpu.VMEM_SHARED((10240,), _f32),
            pltpu.SemaphoreType.DMA,
            pltpu.SemaphoreType.DMA,
            pltpu.SemaphoreType.DMA,
            pltpu.SemaphoreType.DMA,
            pltpu.SemaphoreType.DMA,
        ],
    )


def _sc_roots_body(nchunks, n_pad,
                   didx_hbm, sidx_hbm, cntd_out, cnts_out,
                   jidx_v, iidx_v, ones_v, cntd_sh, cnts_sh, sem):
    c = lax.axis_index("c")
    s = lax.axis_index("s")
    wid = s * _NC + c
    stripe = n_pad // _NS
    nb = stripe // _CH

    if True:
        for k in range(_CH // 16):
            ones_v[pl.ds(k * 16, 16)] = jnp.zeros((16,), _f32)
        for b in range(nb):
            off = s * stripe + b * _CH
            pltpu.sync_copy(ones_v, cntd_sh.at[pl.ds(off, _CH)])
            pltpu.sync_copy(ones_v, cnts_sh.at[pl.ds(off, _CH)])
        for k in range(_CH // 16):
            ones_v[pl.ds(k * 16, 16)] = jnp.ones((16,), _f32)
        plsc.subcore_barrier()

        def chunk(ci, _):
            base = wid * (nchunks * _CH) + ci * _CH
            pltpu.sync_copy(didx_hbm.at[pl.ds(base, _CH)], jidx_v)
            pltpu.sync_copy(sidx_hbm.at[pl.ds(base, _CH)], iidx_v)
            pltpu.sync_copy(ones_v, cntd_sh.at[jidx_v], add=True)
            pltpu.sync_copy(ones_v, cnts_sh.at[iidx_v], add=True)
            return 0

        lax.fori_loop(0, nchunks, chunk, 0)
        plsc.subcore_barrier()
        for b in range(nb):
            off = s * stripe + b * _CH
            pltpu.sync_copy(cntd_sh.at[pl.ds(off, _CH)],
                            cntd_out.at[c, pl.ds(off, _CH)])
            pltpu.sync_copy(cnts_sh.at[pl.ds(off, _CH)],
                            cnts_out.at[c, pl.ds(off, _CH)])


def _make_sc_roots(n_pad, e_pad):
    nchunks = e_pad // (_NW * _CH)
    mesh = plsc.VectorSubcoreMesh(core_axis_name="c", subcore_axis_name="s")
    return pl.kernel(
        functools.partial(_sc_roots_body, nchunks, n_pad),
        out_type=(
            jax.ShapeDtypeStruct((_NC, n_pad), _f32),
            jax.ShapeDtypeStruct((_NC, n_pad), _f32),
        ),
        mesh=mesh,
        scratch_types=[
            pltpu.VMEM((_CH,), jnp.int32),
            pltpu.VMEM((_CH,), jnp.int32),
            pltpu.VMEM((_CH,), _f32),
            pltpu.VMEM_SHARED((10240,), _f32),
            pltpu.VMEM_SHARED((10240,), _f32),
            pltpu.SemaphoreType.DMA,
        ],
    )


def _tc_edge_body(ea_ref, welT_ref, wak_ref, bel_ref, eedot_ref, eemax_ref):
    i = pl.program_id(0)
    wak = wak_ref[...]
    wtil = jnp.dot(welT_ref[...], wak, preferred_element_type=_f32)
    bv = jnp.dot(bel_ref[...], wak, preferred_element_type=_f32)
    ed = jnp.dot(ea_ref[...], wtil, preferred_element_type=_f32) + bv
    eedot_ref[...] = ed

    @pl.when(i == 0)
    def _():
        eemax_ref[0, 0] = -1e30
    eemax_ref[0, 0] = jnp.maximum(eemax_ref[0, 0], jnp.max(ed))


def _tc_edge(ea, welT, wak, bel, blk):
    e = ea.shape[0]
    grid = e // blk
    return pl.pallas_call(
        _tc_edge_body,
        grid=(grid,),
        in_specs=[
            pl.BlockSpec((blk, _D), lambda i: (i, 0)),
            pl.BlockSpec((_D, _D), lambda i: (0, 0)),
            pl.BlockSpec((_D, 1), lambda i: (0, 0)),
            pl.BlockSpec((1, _D), lambda i: (0, 0)),
        ],
        out_specs=[
            pl.BlockSpec((blk, 1), lambda i: (i, 0)),
            pl.BlockSpec((1, 1), lambda i: (0, 0), memory_space=pltpu.SMEM),
        ],
        out_shape=[
            jax.ShapeDtypeStruct((e, 1), _f32),
            jax.ShapeDtypeStruct((1, 1), _f32),
        ],
    )(ea, welT, wak, bel)


def _tc_sweep_body(hp_ref, wih_ref, bih_ref, bhh_ref, waq_ref, wak_ref,
                   cnt_ref, gi_ref, qdot_ref, qmax_ref, h0_ref, hdot_ref,
                   hmax_ref):
    i = pl.program_id(0)
    hp = hp_ref[...]
    gi = jnp.dot(hp, wih_ref[...], preferred_element_type=_f32) + bih_ref[...]
    gi_ref[...] = gi
    qd = jnp.dot(hp, waq_ref[...], preferred_element_type=_f32)
    qdot_ref[...] = qd
    bhh = bhh_ref[...]
    r = jax.nn.sigmoid(gi[:, :_D] + bhh[:, :_D])
    z = jax.nn.sigmoid(gi[:, _D:2 * _D] + bhh[:, _D:2 * _D])
    n = jnp.tanh(gi[:, 2 * _D:] + r * bhh[:, 2 * _D:])
    init = (1.0 - z) * n
    root = cnt_ref[...] == 0.0
    h0 = jnp.where(root, init, 0.0)
    h0_ref[...] = h0
    hd = jnp.dot(h0, wak_ref[...], preferred_element_type=_f32)
    hdot_ref[...] = hd

    @pl.when(i == 0)
    def _():
        qmax_ref[0, 0] = -1e30
        hmax_ref[0, 0] = -1e30
    qmax_ref[0, 0] = jnp.maximum(qmax_ref[0, 0], jnp.max(qd))
    hmax_ref[0, 0] = jnp.maximum(hmax_ref[0, 0], jnp.max(hd))


def _tc_sweep(hp, wihT, bih, bhh, waq, wak, cnt, blk):
    n = hp.shape[0]
    grid = n // blk
    return pl.pallas_call(
        _tc_sweep_body,
        grid=(grid,),
        in_specs=[
            pl.BlockSpec((blk, _D), lambda i: (i, 0)),
            pl.BlockSpec((_D, 3 * _D), lambda i: (0, 0)),
            pl.BlockSpec((1, 3 * _D), lambda i: (0, 0)),
            pl.BlockSpec((1, 3 * _D), lambda i: (0, 0)),
            pl.BlockSpec((_D, 1), lambda i: (0, 0)),
            pl.BlockSpec((_D, 1), lambda i: (0, 0)),
            pl.BlockSpec((blk, 1), lambda i: (i, 0)),
        ],
        out_specs=[
            pl.BlockSpec((blk, 3 * _D), lambda i: (i, 0)),
            pl.BlockSpec((blk, 1), lambda i: (i, 0)),
            pl.BlockSpec((1, 1), lambda i: (0, 0), memory_space=pltpu.SMEM),
            pl.BlockSpec((blk, _D), lambda i: (i, 0)),
            pl.BlockSpec((blk, 1), lambda i: (i, 0)),
            pl.BlockSpec((1, 1), lambda i: (0, 0), memory_space=pltpu.SMEM),
        ],
        out_shape=[
            jax.ShapeDtypeStruct((n, 3 * _D), _f32),
            jax.ShapeDtypeStruct((n, 1), _f32),
            jax.ShapeDtypeStruct((1, 1), _f32),
            jax.ShapeDtypeStruct((n, _D), _f32),
            jax.ShapeDtypeStruct((n, 1), _f32),
            jax.ShapeDtypeStruct((1, 1), _f32),
        ],
    )(hp, wihT, bih, bhh, waq, wak, cnt)


def _tc_level_body(gi_ref, m0_ref, m1_ref, d0_ref, d1_ref, hc_ref, whh_ref,
                   bhh_ref, wak_ref, h_ref, hdot_ref, hmax_ref):
    i = pl.program_id(0)
    d = d0_ref[...] + d1_ref[...]
    m = (m0_ref[...] + m1_ref[...]) / (d + 1e-16)
    gi = gi_ref[...]
    gh = jnp.dot(m, whh_ref[...], preferred_element_type=_f32) + bhh_ref[...]
    r = jax.nn.sigmoid(gi[:, :_D] + gh[:, :_D])
    z = jax.nn.sigmoid(gi[:, _D:2 * _D] + gh[:, _D:2 * _D])
    n = jnp.tanh(gi[:, 2 * _D:] + r * gh[:, 2 * _D:])
    upd = (1.0 - z) * n + z * m
    nm = d > 0.0
    h = jnp.where(nm, upd, hc_ref[...])
    h_ref[...] = h
    hd = jnp.dot(h, wak_ref[...], preferred_element_type=_f32)
    hdot_ref[...] = hd

    @pl.when(i == 0)
    def _():
        hmax_ref[0, 0] = -1e30
    hmax_ref[0, 0] = jnp.maximum(hmax_ref[0, 0], jnp.max(hd))


def _tc_level(gi, m0, m1, d0, d1, hc, whhT, bhh, wak, blk):
    n = hc.shape[0]
    grid = n // blk
    return pl.pallas_call(
        _tc_level_body,
        grid=(grid,),
        in_specs=[
            pl.BlockSpec((blk, 3 * _D), lambda i: (i, 0)),
            pl.BlockSpec((blk, _D), lambda i: (i, 0)),
            pl.BlockSpec((blk, _D), lambda i: (i, 0)),
            pl.BlockSpec((blk, 1), lambda i: (i, 0)),
            pl.BlockSpec((blk, 1), lambda i: (i, 0)),
            pl.BlockSpec((blk, _D), lambda i: (i, 0)),
            pl.BlockSpec((_D, 3 * _D), lambda i: (0, 0)),
            pl.BlockSpec((1, 3 * _D), lambda i: (0, 0)),
            pl.BlockSpec((_D, 1), lambda i: (0, 0)),
        ],
        out_specs=[
            pl.BlockSpec((blk, _D), lambda i: (i, 0)),
            pl.BlockSpec((blk, 1), lambda i: (i, 0)),
            pl.BlockSpec((1, 1), lambda i: (0, 0), memory_space=pltpu.SMEM),
        ],
        out_shape=[
            jax.ShapeDtypeStruct((n, _D), _f32),
            jax.ShapeDtypeStruct((n, 1), _f32),
            jax.ShapeDtypeStruct((1, 1), _f32),
        ],
    )(gi, m0, m1, d0, d1, hc, whhT, bhh, wak)


def _tc_relu_body(x_ref, o_ref):
    o_ref[...] = jnp.maximum(x_ref[...], 0.0)


def _tc_relu(x, blk):
    n = x.shape[0]
    return pl.pallas_call(
        _tc_relu_body,
        grid=(n // blk,),
        in_specs=[pl.BlockSpec((blk, _D), lambda i: (i, 0))],
        out_specs=pl.BlockSpec((blk, _D), lambda i: (i, 0)),
        out_shape=jax.ShapeDtypeStruct(x.shape, x.dtype),
    )(x)


def kernel(x, edge_index, edge_attr, edge_masks,
           Wel_f, bel_f, Wa_f, ba_f, Wih_f, Whh_f, bih_f, bhh_f,
           Wel_b, bel_b, Wa_b, ba_b, Wih_b, Whh_b, bih_b, bhh_b):
    n, d = x.shape
    e = edge_index.shape[1]
    nl = edge_masks.shape[0]
    el = e // nl

    n_pad = _cdiv(n, _NS * _CH) * (_NS * _CH)
    e_lvl = _cdiv(el, _NW * _CH) * (_NW * _CH)
    e_all = _cdiv(e, _NW * _CH) * (_NW * _CH)
    blk = n_pad // 10

    src = edge_index[0]
    dst = edge_index[1]

    pad_l = e_lvl - el
    src_l = jnp.pad(src.reshape(nl, el), ((0, 0), (0, pad_l)))
    src_l = src_l.reshape(nl, _NW, e_lvl // (_NW * _CH), _CH)
    dst_l = jnp.pad(dst.reshape(nl, el), ((0, 0), (0, pad_l)))
    dst_l = dst_l.reshape(nl, _NW, e_lvl // (_NW * _CH), _CH)
    zrow = jnp.zeros((n_pad, d), _f32)
    zden = jnp.zeros((n_pad,), _f32)

    pad_a = e_all - e
    src_a = jnp.pad(src, (0, pad_a), constant_values=n_pad - 1)
    dst_a = jnp.pad(dst, (0, pad_a), constant_values=n_pad - 1)

    x_pad = jnp.pad(x, ((0, n_pad - n), (0, 0)))
    e_blk = _cdiv(e, 1024) * 1024
    ea_pad = jnp.pad(edge_attr, ((0, e_blk - e), (0, 0)))

    sc_attn = _make_sc_attn(n_pad, e_lvl)
    sc_roots = _make_sc_roots(n_pad, e_all)

    cntd2, cnts2 = sc_roots(dst_a, src_a)
    cntd = (cntd2[0] + cntd2[1]).reshape(n_pad, 1)
    cnts = (cnts2[0] + cnts2[1]).reshape(n_pad, 1)

    def sweep(h_prev, cnt, j_l, i_l, order,
              Wel, bel, Wa, ba, Wih, Whh, bih, bhh):
        waq = Wa[0, :d].reshape(d, 1)
        wak = Wa[0, d:].reshape(d, 1)
        eedot, eemax = _tc_edge(ea_pad, Wel.T, wak, bel.reshape(1, d),
                                blk=1024)
        ee_l = jnp.pad(eedot[:e, 0].reshape(nl, el), ((0, 0), (0, pad_l)),
                       constant_values=-1e30)
        gi, qdot, qmax, h, hdot, hmax = _tc_sweep(
            h_prev, Wih.T, bih.reshape(1, 3 * d), bhh.reshape(1, 3 * d),
            waq, wak, cnt, blk)
        whhT = Whh.T
        bhh2 = bhh.reshape(1, 3 * d)
        for l in order:
            shift = ba[0] - qmax[0, 0] - hmax[0, 0] - eemax[0, 0]
            shift_v = jnp.full((16,), shift, _f32)
            msg2, den2 = sc_attn(qdot.reshape(n_pad), hdot.reshape(n_pad),
                                 ee_l[l], j_l[l], i_l[l], h, shift_v,
                                 zrow, zden)
            h, hdot, hmax = _tc_level(
                gi, msg2[0], msg2[1], den2[0].reshape(n_pad, 1),
                den2[1].reshape(n_pad, 1), h, whhT, bhh2, wak, blk)
        return h

    h_fwd = sweep(x_pad, cntd, src_l, dst_l, range(nl),
                  Wel_f, bel_f, Wa_f, ba_f, Wih_f, Whh_f, bih_f, bhh_f)
    h_bwd = sweep(h_fwd, cnts, dst_l, src_l, range(nl - 1, -1, -1),
                  Wel_b, bel_b, Wa_b, ba_b, Wih_b, Whh_b, bih_b, bhh_b)
    return _tc_relu(h_bwd, blk)[:n]
